# baseline (speedup 1.0000x reference)
import numpy as np

import jax
import jax.numpy as jnp
from jax.experimental import pallas as pl
from jax.sharding import Mesh, PartitionSpec as P


def _matmul_block(x_ref, w_ref, o_ref):
    o_ref[...] = jax.lax.dot_general(
        x_ref[...],
        w_ref[...],
        dimension_numbers=(((1,), (1,)), ((), ())),
        preferred_element_type=jnp.float32,
    )


def _router_logits(hidden_states, W):
    T, K = hidden_states.shape
    N = W.shape[0]
    bm = min(1024, T)
    return pl.pallas_call(
        _matmul_block,
        grid=(T // bm,),
        in_specs=[
            pl.BlockSpec((bm, K), lambda i: (i, 0)),
            pl.BlockSpec((N, K), lambda i: (0, 0)),
        ],
        out_specs=pl.BlockSpec((bm, N), lambda i: (i, 0)),
        out_shape=jax.ShapeDtypeStruct((T, N), jnp.float32),
    )(hidden_states, W)


def kernel(hidden_states, W):
    devs = jax.devices()
    n = len(devs)
    if hidden_states.shape[0] % max(n, 1) != 0 or n <= 1:
        return _router_logits(hidden_states, W)
    mesh = Mesh(np.array(devs), ("x",))
    sharded = jax.shard_map(
        _router_logits,
        mesh=mesh,
        in_specs=(P("x", None), P(None, None)),
        out_specs=P("x", None),
        check_vma=False,
    )
    return sharded(hidden_states, W)


# speedup vs baseline: 6.6259x; 6.6259x over previous
import functools

import jax
import jax.numpy as jnp
from jax.experimental import pallas as pl


def _matmul_block(x_ref, w_ref, o_ref):
    o_ref[...] = jax.lax.dot_general(
        x_ref[...],
        w_ref[...],
        dimension_numbers=(((1,), (1,)), ((), ())),
        preferred_element_type=jnp.float32,
    )


@functools.partial(jax.jit, static_argnames=("bm",))
def _router_logits(hidden_states, W, bm=1024):
    T, K = hidden_states.shape
    N = W.shape[0]
    return pl.pallas_call(
        _matmul_block,
        grid=(T // bm,),
        in_specs=[
            pl.BlockSpec((bm, K), lambda i: (i, 0)),
            pl.BlockSpec((N, K), lambda i: (0, 0)),
        ],
        out_specs=pl.BlockSpec((bm, N), lambda i: (i, 0)),
        out_shape=jax.ShapeDtypeStruct((T, N), jnp.float32),
    )(hidden_states, W)


def kernel(hidden_states, W):
    return _router_logits(hidden_states, W)
